# baseline (speedup 1.0000x reference)
import functools

import jax
import jax.numpy as jnp
from jax import lax
from jax.experimental import pallas as pl
from jax.experimental.pallas import tpu as pltpu
from jax.experimental.pallas import tpu_sc as plsc

_B = 4
_N = 100000
_D = 64
_W = 2 * _D
_NPAIR = _N // 2
_NP = 50
_NPAD = 64
_NU = _B * _NPAD
_NWORK = 32
_UPW = _NU // _NWORK
_BLK = 2000


def _sc_vals_body(tab_hbm, f2_hbm, aw_hbm, narr2_hbm, vout_hbm,
                  trows, frows, aw_v, narrh, usem):
    cid = lax.axis_index("c")
    sid = lax.axis_index("s")
    wid = sid * 2 + cid
    ub = wid * _UPW

    pltpu.sync_copy(narr2_hbm.at[pl.ds(ub, _UPW)], narrh)
    pltpu.sync_copy(f2_hbm.at[pl.ds(ub, _UPW)], frows)
    pltpu.sync_copy(aw_hbm.at[pl.ds(ub, _UPW)], aw_v)
    pltpu.make_async_copy(tab_hbm.at[narrh], trows, usem).start()
    pltpu.make_async_copy(tab_hbm.at[narrh], trows, usem).wait()

    def row(i, carry):
        for k in range(_W // 16):
            a = aw_v[i, pl.ds((k // 4) * 16, 16)]
            t = trows[i, pl.ds(k * 16, 16)]
            f = frows[i, pl.ds(k * 16, 16)]
            trows[i, pl.ds(k * 16, 16)] = t + a * (f - t)
        return carry

    lax.fori_loop(0, _UPW, row, 0)
    pltpu.sync_copy(trows, vout_hbm.at[pl.ds(ub, _UPW)])


_NBLK = _NPAIR // _BLK


def _tc_body(tab_ref, narr2_ref, vals_ref, out_ref):
    i = pl.program_id(0)
    g = i // _NBLK
    base = (i % _NBLK) * _BLK
    tab = tab_ref[...]
    prow = jax.lax.broadcasted_iota(jnp.int32, (_BLK, _NPAD), 0) + base
    ii = jax.lax.broadcasted_iota(jnp.int32, (_NPAD, _NPAD), 0)
    jj = jax.lax.broadcasted_iota(jnp.int32, (_NPAD, _NPAD), 1)
    later = jj > ii
    gsel = jax.lax.broadcasted_iota(jnp.int32, (8, 1), 0) == g
    ng = jnp.max(jnp.where(gsel, narr2_ref[...],
                           jnp.int32(-2147483648)), axis=0, keepdims=True)
    ngc = ng.reshape(_NPAD, 1)
    dup = jnp.any((ngc == ng) & later, axis=1, keepdims=True)
    kept = jnp.logical_not(dup).reshape(1, _NPAD)
    onehot = ((prow == ng) & kept).astype(jnp.float32)
    betak = jnp.max(onehot, axis=1, keepdims=True)
    vals_g = vals_ref[pl.ds(pl.multiple_of(g * _NPAD, _NPAD), _NPAD), :]
    upd = jnp.dot(onehot, vals_g,
                  preferred_element_type=jnp.float32)
    out_ref[...] = tab * (1.0 - betak) + upd


@jax.jit
def _run(tab2, f2, aw, narr2, narr2_grid):
    mesh = plsc.VectorSubcoreMesh(core_axis_name="c", subcore_axis_name="s")
    vals = functools.partial(
        pl.kernel,
        out_type=jax.ShapeDtypeStruct((_NU, _W), jnp.float32),
        mesh=mesh,
        scratch_types=[
            pltpu.VMEM((_UPW, _W), jnp.float32),
            pltpu.VMEM((_UPW, _W), jnp.float32),
            pltpu.VMEM((_UPW, 32), jnp.float32),
            pltpu.VMEM((_UPW,), jnp.int32),
            pltpu.SemaphoreType.DMA,
        ],
    )(_sc_vals_body)(tab2, f2, aw, narr2)

    out = pl.pallas_call(
        _tc_body,
        grid=(_B * _NBLK,),
        in_specs=[
            pl.BlockSpec((_BLK, _W), lambda i: (i % _NBLK, 0)),
            pl.BlockSpec((8, _NPAD), lambda i: (0, 0)),
            pl.BlockSpec((_NU, _W), lambda i: (0, 0)),
        ],
        out_specs=pl.BlockSpec((_BLK, _W), lambda i: (i, 0)),
        out_shape=jax.ShapeDtypeStruct((_B * _NPAIR, _W), jnp.float32),
    )(tab2, narr2_grid, vals)
    return out.reshape(_B, _N, _D)


def kernel(nodes_output, item_table, alpha, nodes, batch_num_nodes):
    nodes2d = nodes.reshape(_B, _NP)
    padc = jnp.broadcast_to(nodes2d[:, -1:], (_B, _NPAD - _NP))
    nodes_pad = jnp.concatenate([nodes2d, padc], axis=1)
    eq = nodes_pad[:, :, None] == nodes_pad[:, None, :]
    jidx = jnp.arange(_NPAD, dtype=jnp.int32)[None, None, :]
    lastocc = jnp.max(jnp.where(eq, jidx, -1), axis=2)
    feat = nodes_output.reshape(_B, _NP, _D)
    fpad = jnp.broadcast_to(feat[:, -1:, :], (_B, _NPAD - _NP, _D))
    feat_pad = jnp.concatenate([feat, fpad], axis=1)
    feat_eff = jnp.take_along_axis(feat_pad, lastocc[:, :, None], axis=1)
    feat_eff = feat_eff.reshape(_NU, _D)

    narr = nodes_pad.reshape(_NU)
    av = alpha.reshape(_N)[narr]
    par = narr & 1

    mate_eq = nodes_pad[:, :, None] == (nodes_pad[:, None, :] ^ 1)
    has_mate = jnp.any(mate_eq, axis=2).reshape(_NU)
    mate_loc = jnp.argmax(mate_eq, axis=2)
    mate_idx = (mate_loc
                + (jnp.arange(_B, dtype=jnp.int32) * _NPAD)[:, None]
                ).reshape(_NU)
    f_mate = feat_eff[mate_idx]
    a_mate = jnp.where(has_mate, av[mate_idx], 0.0)

    sel = (par == 0)[:, None]
    fhalf0 = jnp.where(sel, feat_eff, f_mate)
    fhalf1 = jnp.where(sel, f_mate, feat_eff)
    ahalf0 = jnp.where(par == 0, av, a_mate)
    ahalf1 = jnp.where(par == 0, a_mate, av)
    f2 = jnp.concatenate([fhalf0, fhalf1], axis=1)
    aw = jnp.concatenate(
        [jnp.broadcast_to(ahalf0[:, None], (_NU, 16)),
         jnp.broadcast_to(ahalf1[:, None], (_NU, 16))], axis=1)

    narr2 = narr >> 1
    narr2_grid = jnp.concatenate(
        [narr2.reshape(_B, _NPAD),
         jnp.zeros((8 - _B, _NPAD), jnp.int32) - 1], axis=0)

    return _run(item_table.reshape(_NPAIR, _W), f2, aw, narr2, narr2_grid)


# speedup vs baseline: 1.3054x; 1.3054x over previous
import functools

import jax
import jax.numpy as jnp
from jax import lax
from jax.experimental import pallas as pl
from jax.experimental.pallas import tpu as pltpu
from jax.experimental.pallas import tpu_sc as plsc

_B = 4
_N = 100000
_D = 64
_W = 2 * _D
_NPAIR = _N // 2
_NP = 50
_NPAD = 64
_NU = _B * _NPAD
_NWORK = 32
_UPW = _NU // _NWORK
_BLK = 2000


def _sc_vals_body(tab_hbm, f2_hbm, aw_hbm, narr2_hbm, vout_hbm,
                  trows, frows, aw_v, narrh, usem):
    cid = lax.axis_index("c")
    sid = lax.axis_index("s")
    wid = sid * 2 + cid
    ub = wid * _UPW

    pltpu.sync_copy(narr2_hbm.at[pl.ds(ub, _UPW)], narrh)
    pltpu.sync_copy(f2_hbm.at[pl.ds(ub, _UPW)], frows)
    pltpu.sync_copy(aw_hbm.at[pl.ds(ub, _UPW)], aw_v)
    pltpu.make_async_copy(tab_hbm.at[narrh], trows, usem).start()
    pltpu.make_async_copy(tab_hbm.at[narrh], trows, usem).wait()

    def row(i, carry):
        for k in range(_W // 16):
            a = aw_v[i, pl.ds((k // 4) * 16, 16)]
            t = trows[i, pl.ds(k * 16, 16)]
            f = frows[i, pl.ds(k * 16, 16)]
            trows[i, pl.ds(k * 16, 16)] = t + a * (f - t)
        return carry

    lax.fori_loop(0, _UPW, row, 0)
    pltpu.sync_copy(trows, vout_hbm.at[pl.ds(ub, _UPW)])


_RBLK = 4000


def _tc_body(tab_ref, nodes_ref, vals_ref, par_ref, out_ref):
    base = pl.program_id(0) * _RBLK
    tab = tab_ref[...]
    sel = par_ref[...] > 0
    rowvals = jnp.where(sel, vals_ref[:, _D:], vals_ref[:, :_D])
    rows = jax.lax.broadcasted_iota(jnp.int32, (_RBLK, _NPAD), 0) + base
    ii = jax.lax.broadcasted_iota(jnp.int32, (_NPAD, _NPAD), 0)
    jj = jax.lax.broadcasted_iota(jnp.int32, (_NPAD, _NPAD), 1)
    later = jj > ii
    for g in range(_B):
        ng = nodes_ref[g, :].reshape(1, _NPAD)
        ngc = ng.reshape(_NPAD, 1)
        dup = jnp.any((ngc == ng) & later, axis=1, keepdims=True)
        kept = jnp.logical_not(dup).reshape(1, _NPAD)
        onehot = ((rows == ng) & kept).astype(jnp.float32)
        betak = jnp.max(onehot, axis=1, keepdims=True)
        upd = jnp.dot(onehot, rowvals[g * _NPAD:(g + 1) * _NPAD, :],
                      preferred_element_type=jnp.float32)
        out_ref[g] = tab * (1.0 - betak) + upd


@jax.jit
def _run(tab2, f2, aw, narr2, nodes_grid, parw):
    mesh = plsc.VectorSubcoreMesh(core_axis_name="c", subcore_axis_name="s")
    vals = functools.partial(
        pl.kernel,
        out_type=jax.ShapeDtypeStruct((_NU, _W), jnp.float32),
        mesh=mesh,
        scratch_types=[
            pltpu.VMEM((_UPW, _W), jnp.float32),
            pltpu.VMEM((_UPW, _W), jnp.float32),
            pltpu.VMEM((_UPW, 32), jnp.float32),
            pltpu.VMEM((_UPW,), jnp.int32),
            pltpu.SemaphoreType.DMA,
        ],
    )(_sc_vals_body)(tab2, f2, aw, narr2)

    out = pl.pallas_call(
        _tc_body,
        grid=(_N // _RBLK,),
        in_specs=[
            pl.BlockSpec((_RBLK, _D), lambda i: (i, 0)),
            pl.BlockSpec((8, _NPAD), lambda i: (0, 0)),
            pl.BlockSpec((_NU, _W), lambda i: (0, 0)),
            pl.BlockSpec((_NU, 1), lambda i: (0, 0)),
        ],
        out_specs=pl.BlockSpec((_B, _RBLK, _D), lambda i: (0, i, 0)),
        out_shape=jax.ShapeDtypeStruct((_B, _N, _D), jnp.float32),
    )(tab2.reshape(_N, _D), nodes_grid, vals, parw)
    return out


def kernel(nodes_output, item_table, alpha, nodes, batch_num_nodes):
    nodes2d = nodes.reshape(_B, _NP)
    padc = jnp.broadcast_to(nodes2d[:, -1:], (_B, _NPAD - _NP))
    nodes_pad = jnp.concatenate([nodes2d, padc], axis=1)
    eq = nodes_pad[:, :, None] == nodes_pad[:, None, :]
    jidx = jnp.arange(_NPAD, dtype=jnp.int32)[None, None, :]
    lastocc = jnp.max(jnp.where(eq, jidx, -1), axis=2)
    feat = nodes_output.reshape(_B, _NP, _D)
    fpad = jnp.broadcast_to(feat[:, -1:, :], (_B, _NPAD - _NP, _D))
    feat_pad = jnp.concatenate([feat, fpad], axis=1)
    feat_eff = jnp.take_along_axis(feat_pad, lastocc[:, :, None], axis=1)
    feat_eff = feat_eff.reshape(_NU, _D)

    narr = nodes_pad.reshape(_NU)
    av = alpha.reshape(_N)[narr]
    par = narr & 1

    mate_eq = nodes_pad[:, :, None] == (nodes_pad[:, None, :] ^ 1)
    has_mate = jnp.any(mate_eq, axis=2).reshape(_NU)
    mate_loc = jnp.argmax(mate_eq, axis=2)
    mate_idx = (mate_loc
                + (jnp.arange(_B, dtype=jnp.int32) * _NPAD)[:, None]
                ).reshape(_NU)
    f_mate = feat_eff[mate_idx]
    a_mate = jnp.where(has_mate, av[mate_idx], 0.0)

    sel = (par == 0)[:, None]
    fhalf0 = jnp.where(sel, feat_eff, f_mate)
    fhalf1 = jnp.where(sel, f_mate, feat_eff)
    ahalf0 = jnp.where(par == 0, av, a_mate)
    ahalf1 = jnp.where(par == 0, a_mate, av)
    f2 = jnp.concatenate([fhalf0, fhalf1], axis=1)
    aw = jnp.concatenate(
        [jnp.broadcast_to(ahalf0[:, None], (_NU, 16)),
         jnp.broadcast_to(ahalf1[:, None], (_NU, 16))], axis=1)

    narr2 = narr >> 1
    nodes_grid = jnp.concatenate(
        [nodes_pad, jnp.zeros((8 - _B, _NPAD), jnp.int32) - 1], axis=0)
    parw = par.astype(jnp.float32).reshape(_NU, 1)

    return _run(item_table.reshape(_NPAIR, _W), f2, aw, narr2,
                nodes_grid, parw)
